# baseline (speedup 1.0000x reference)
import functools

import jax
import jax.numpy as jnp
from jax.experimental import pallas as pl
from jax.experimental.pallas import tpu as pltpu

_N = 10000
_E = 160000
_EDIM = 16
_B = 256
_NC = _E // _B


def _mm_kernel(a_ref, b_ref, o_ref):
    o_ref[:] = jnp.dot(a_ref[:], b_ref[:], preferred_element_type=jnp.float32)


def _mm(a, b):
    return pl.pallas_call(
        _mm_kernel,
        out_shape=jax.ShapeDtypeStruct((a.shape[0], b.shape[1]), jnp.float32),
    )(a, b)


def _alpha_kernel(H, C, src_ref, dst_ref, ea_ref, xl_ref, xr_ref, we_ref,
                  attf_ref, alpha_ref, amax_ref, ew_ref):
    @pl.when(pl.program_id(0) == 0)
    def _():
        amax_ref[:] = jnp.full_like(amax_ref, -1e30)

    ew_ref[:] = jnp.dot(ea_ref[:], we_ref[:], preferred_element_type=jnp.float32)

    def body(e, carry):
        s = src_ref[e]
        d = dst_ref[e]
        xj = xl_ref[pl.ds(s, 1), :]
        xi = xr_ref[pl.ds(d, 1), :]
        v = xi + xj + ew_ref[pl.ds(e, 1), :]
        z = jnp.where(v >= 0.0, v, 0.2 * v)
        za = z * attf_ref[:]
        for h in range(H):
            ah = jnp.sum(za[:, h * C:(h + 1) * C], axis=1, keepdims=True)
            alpha_ref[pl.ds(e, 1), pl.ds(h, 1)] = ah
            cur = amax_ref[pl.ds(d, 1), pl.ds(h, 1)]
            amax_ref[pl.ds(d, 1), pl.ds(h, 1)] = jnp.maximum(cur, ah)
        return carry

    jax.lax.fori_loop(0, _B, body, 0)


def _denom_kernel(H, src_ref, dst_ref, alpha_ref, amax_ref, ex_ref, denom_ref):
    @pl.when(pl.program_id(0) == 0)
    def _():
        denom_ref[:] = jnp.zeros_like(denom_ref)

    def body(e, carry):
        d = dst_ref[e]
        for h in range(H):
            a = alpha_ref[pl.ds(e, 1), pl.ds(h, 1)]
            m = amax_ref[pl.ds(d, 1), pl.ds(h, 1)]
            exv = jnp.exp(a - m)
            ex_ref[pl.ds(e, 1), pl.ds(h, 1)] = exv
            denom_ref[pl.ds(d, 1), pl.ds(h, 1)] = (
                denom_ref[pl.ds(d, 1), pl.ds(h, 1)] + exv)
        return carry

    jax.lax.fori_loop(0, _B, body, 0)


def _agg_kernel(H, C, src_ref, dst_ref, ex_ref, denom_ref, xl_ref, b_ref,
                out_ref):
    @pl.when(pl.program_id(0) == 0)
    def _():
        out_ref[:] = jnp.zeros_like(out_ref)

    def body(e, carry):
        s = src_ref[e]
        d = dst_ref[e]
        for h in range(H):
            a = ex_ref[pl.ds(e, 1), pl.ds(h, 1)] / (
                denom_ref[pl.ds(d, 1), pl.ds(h, 1)] + 1e-16)
            row = xl_ref[pl.ds(s, 1), h * C:(h + 1) * C]
            out_ref[pl.ds(d, 1), h * C:(h + 1) * C] = (
                out_ref[pl.ds(d, 1), h * C:(h + 1) * C] + a * row)
        return carry

    jax.lax.fori_loop(0, _B, body, 0)

    @pl.when(pl.program_id(0) == _NC - 1)
    def _():
        out_ref[:] = jnp.maximum(out_ref[:] + b_ref[:], 0.0)


def _gat_layer(x, src, dst, edge_attr, Wl, Wr, We, att, bias, H, C):
    HC = H * C
    xl = _mm(x, Wl)
    xr = _mm(x, Wr)
    attf = att.reshape(1, HC)
    bf = bias.reshape(1, HC)

    idx_spec = pl.BlockSpec((_B,), lambda i: (i,), memory_space=pltpu.SMEM)
    full_n = pl.BlockSpec((_N, HC), lambda i: (0, 0))
    full_nh = pl.BlockSpec((_N, H), lambda i: (0, 0))
    chunk_eh = pl.BlockSpec((_B, H), lambda i: (i, 0))

    alpha, amax = pl.pallas_call(
        functools.partial(_alpha_kernel, H, C),
        grid=(_NC,),
        in_specs=[
            idx_spec,
            idx_spec,
            pl.BlockSpec((_B, _EDIM), lambda i: (i, 0)),
            full_n,
            full_n,
            pl.BlockSpec((_EDIM, HC), lambda i: (0, 0)),
            pl.BlockSpec((1, HC), lambda i: (0, 0)),
        ],
        out_specs=[chunk_eh, full_nh],
        out_shape=[
            jax.ShapeDtypeStruct((_E, H), jnp.float32),
            jax.ShapeDtypeStruct((_N, H), jnp.float32),
        ],
        scratch_shapes=[pltpu.VMEM((_B, HC), jnp.float32)],
    )(src, dst, edge_attr, xl, xr, We, attf)

    ex, denom = pl.pallas_call(
        functools.partial(_denom_kernel, H),
        grid=(_NC,),
        in_specs=[idx_spec, idx_spec, chunk_eh, full_nh],
        out_specs=[chunk_eh, full_nh],
        out_shape=[
            jax.ShapeDtypeStruct((_E, H), jnp.float32),
            jax.ShapeDtypeStruct((_N, H), jnp.float32),
        ],
    )(src, dst, alpha, amax)

    out = pl.pallas_call(
        functools.partial(_agg_kernel, H, C),
        grid=(_NC,),
        in_specs=[
            idx_spec,
            idx_spec,
            chunk_eh,
            full_nh,
            full_n,
            pl.BlockSpec((1, HC), lambda i: (0, 0)),
        ],
        out_specs=full_n,
        out_shape=jax.ShapeDtypeStruct((_N, HC), jnp.float32),
    )(src, dst, ex, denom, xl, bf)
    return out


def _head_kernel(h_ref, w1_ref, b1_ref, w2_ref, b2_ref, o_ref):
    d = jnp.dot(h_ref[0:1, :], w1_ref[:], preferred_element_type=jnp.float32) + b1_ref[:]
    o_ref[:] = jnp.tanh(
        jnp.dot(d, w2_ref[:], preferred_element_type=jnp.float32) + b2_ref[:])


def kernel(x, edge_index, edge_attr, W1l, W1r, W1e, att1, b1, W2l, W2r, W2e,
           att2, b2, D1w, D1b, D2w, D2b):
    src = edge_index[0]
    dst = edge_index[1]
    h1 = _gat_layer(x, src, dst, edge_attr, W1l, W1r, W1e, att1, b1, 2, 256)
    h2 = _gat_layer(h1, src, dst, edge_attr, W2l, W2r, W2e, att2, b2, 1, 512)

    out = pl.pallas_call(
        _head_kernel,
        grid=(1,),
        in_specs=[
            pl.BlockSpec((8, 512), lambda i: (0, 0)),
            pl.BlockSpec((512, _N), lambda i: (0, 0)),
            pl.BlockSpec((1, _N), lambda i: (0, 0)),
            pl.BlockSpec((_N, 64), lambda i: (0, 0)),
            pl.BlockSpec((1, 64), lambda i: (0, 0)),
        ],
        out_specs=pl.BlockSpec((1, 64), lambda i: (0, 0)),
        out_shape=jax.ShapeDtypeStruct((1, 64), jnp.float32),
    )(h2, D1w, D1b.reshape(1, _N), D2w, D2b.reshape(1, 64))
    return out.reshape(64)
